# baseline (speedup 1.0000x reference)
import functools

import jax
import jax.numpy as jnp
from jax.experimental import pallas as pl
from jax.experimental.pallas import tpu as pltpu

NUM_LAYERS = 4
N_NODES = 10000
D = 256


def _embed_body(f_ref, w_ref, b_ref, o_ref):
    o_ref[...] = (
        jax.lax.dot_general(
            f_ref[0], w_ref[0], (((1,), (1,)), ((), ())),
            preferred_element_type=jnp.float32,
        )[None]
        + b_ref[...]
    )


def _embed(feat, W, b):
    return pl.pallas_call(
        _embed_body,
        grid=(2, 5),
        in_specs=[
            pl.BlockSpec((1, 1000, 256), lambda i, j: (i, j, 0)),
            pl.BlockSpec((1, 256, 256), lambda i, j: (i, 0, 0)),
            pl.BlockSpec((1, 1, 256), lambda i, j: (i, 0, 0)),
        ],
        out_specs=pl.BlockSpec((1, 1000, 256), lambda i, j: (i, j, 0)),
        out_shape=jax.ShapeDtypeStruct((2, 5000, 256), jnp.float32),
    )(feat, W, b)


def kernel(user_feature, movie_feature, edge_index, Wu, bu, Wm, bm):
    feat = jnp.stack([user_feature, movie_feature])
    W = jnp.stack([Wu, Wm])
    b = jnp.stack([bu, bm]).reshape(2, 1, 256)
    x = _embed(feat, W, b).reshape(N_NODES, D)

    row, col = edge_index[0], edge_index[1]
    deg = jnp.zeros((N_NODES,), jnp.float32).at[col].add(1.0)
    dinv = jnp.where(deg > 0, jnp.where(deg > 0, deg, 1.0) ** -0.5, 0.0)
    norm = dinv[row] * dinv[col]

    out = x
    for k in range(NUM_LAYERS):
        msg = norm[:, None] * x[row]
        x = jnp.zeros_like(x).at[col].add(msg)
        out = out + (1.0 / (k + 2)) * x
    return out


# speedup vs baseline: 6.4116x; 6.4116x over previous
import jax
import jax.numpy as jnp
from jax import lax
from jax.experimental import pallas as pl
from jax.experimental.pallas import tpu as pltpu
from jax.experimental.pallas import tpu_sc as plsc

NUM_LAYERS = 4
N_NODES = 10000
NPAD = 10240
D = 256
HALF = 128
NS = 16
NC = 2
STRIPE = NPAD // NS
E_PER_SUB = 160000 // NS
CHUNK = 80
NCHUNK = E_PER_SUB // CHUNK
SUPER = 25
NSUPER = NCHUNK // SUPER


def _embed_body(f_ref, w_ref, b_ref, o_ref):
    o_ref[...] = (
        lax.dot_general(
            f_ref[0], w_ref[0], (((1,), (1,)), ((), ())),
            preferred_element_type=jnp.float32,
        )[None]
        + b_ref[...]
    )


def _embed(feat, W, b):
    return pl.pallas_call(
        _embed_body,
        grid=(2, 5),
        in_specs=[
            pl.BlockSpec((1, 1000, 256), lambda i, j: (i, j, 0)),
            pl.BlockSpec((1, 256, 256), lambda i, j: (i, 0, 0)),
            pl.BlockSpec((1, 1, 256), lambda i, j: (i, 0, 0)),
        ],
        out_specs=pl.BlockSpec((1, 1000, 256), lambda i, j: (i, j, 0)),
        out_shape=jax.ShapeDtypeStruct((2, 5000, 256), jnp.float32),
    )(feat, W, b)


def _sc_body(x0_hbm, er_hbm, ec_hbm,
             x1_hbm, x2_hbm, x3_hbm, x4_hbm, y_hbm,
             acc_sh, deg_sh, rowi_v, coli_v, dinv_v, ones_v,
             zrow_v, zdeg_v, gath_v, buf_a, buf_x, buf_y, sem):
    c = lax.axis_index("c")
    s = lax.axis_index("s")
    my0 = s * STRIPE
    base = c * NPAD + my0
    zero16 = jnp.zeros((16,), jnp.float32)
    one16 = jnp.ones((16,), jnp.float32)
    for i in range(STRIPE // 16):
        zdeg_v[pl.ds(i * 16, 16)] = zero16
    for i in range(16):
        for j in range(HALF // 16):
            zrow_v[i, pl.ds(j * 16, 16)] = zero16
    for i in range(CHUNK // 16):
        ones_v[pl.ds(i * 16, 16)] = one16

    pltpu.sync_copy(zdeg_v, deg_sh.at[pl.ds(my0, STRIPE)])
    plsc.subcore_barrier()

    def deg_sup(g, carry):
        pltpu.sync_copy(ec_hbm.at[s, g], coli_v)

        def deg_step(j, inner):
            pltpu.sync_copy(ones_v, deg_sh.at[coli_v.at[j]], add=True)
            return inner
        lax.fori_loop(0, SUPER, deg_step, 0)
        return carry
    lax.fori_loop(0, NSUPER, deg_sup, 0)
    plsc.subcore_barrier()

    pltpu.sync_copy(deg_sh.at[pl.ds(my0, STRIPE)], dinv_v)
    for i in range(STRIPE // 16):
        d = dinv_v[pl.ds(i * 16, 16)]
        bits = lax.bitcast_convert_type(d, jnp.int32)
        y = lax.bitcast_convert_type(jnp.int32(0x5F3759DF) - (bits >> 1), jnp.float32)
        for _ in range(3):
            y = y * (1.5 - 0.5 * d * y * y)
        dinv_v[pl.ds(i * 16, 16)] = jnp.where(d > 0, y, 0.0)

    def y0_step(k, carry):
        r0 = base + k * 16
        pltpu.sync_copy(x0_hbm.at[pl.ds(r0, 16)], buf_x)
        dv = dinv_v[pl.ds(k * 16, 16)]
        for r in range(16):
            d = dv[r]
            for jj in range(HALF // 16):
                buf_y[r, pl.ds(jj * 16, 16)] = buf_x[r, pl.ds(jj * 16, 16)] * d
        pltpu.sync_copy(buf_y, y_hbm.at[pl.ds(r0, 16)])
        return carry
    lax.fori_loop(0, STRIPE // 16, y0_step, 0)

    outs = [x1_hbm, x2_hbm, x3_hbm, x4_hbm]
    for ell in range(NUM_LAYERS):
        def z_step(k, carry):
            pltpu.sync_copy(zrow_v, acc_sh.at[pl.ds(my0 + k * 16, 16)])
            return carry
        lax.fori_loop(0, STRIPE // 16, z_step, 0)
        plsc.subcore_barrier()

        def gs_sup(g, carry):
            pltpu.sync_copy(er_hbm.at[c, s, g], rowi_v)
            pltpu.sync_copy(ec_hbm.at[s, g], coli_v)

            def gs_step(j, inner):
                pltpu.async_copy(y_hbm.at[rowi_v.at[j]], gath_v, sem).wait()
                pltpu.sync_copy(gath_v, acc_sh.at[coli_v.at[j]], add=True)
                return inner
            lax.fori_loop(0, SUPER, gs_step, 0)
            return carry
        lax.fori_loop(0, NSUPER, gs_sup, 0)
        plsc.subcore_barrier()

        out_hbm = outs[ell]

        def o_step(k, carry):
            pltpu.sync_copy(acc_sh.at[pl.ds(my0 + k * 16, 16)], buf_a)
            dv = dinv_v[pl.ds(k * 16, 16)]
            for r in range(16):
                d = dv[r]
                d2 = d * d
                for jj in range(HALF // 16):
                    a = buf_a[r, pl.ds(jj * 16, 16)]
                    buf_x[r, pl.ds(jj * 16, 16)] = a * d
                    buf_y[r, pl.ds(jj * 16, 16)] = a * d2
            r0 = base + k * 16
            pltpu.sync_copy(buf_x, out_hbm.at[pl.ds(r0, 16)])
            pltpu.sync_copy(buf_y, y_hbm.at[pl.ds(r0, 16)])
            return carry
        lax.fori_loop(0, STRIPE // 16, o_step, 0)


def _propagate(x0_flat, er2, ec3):
    mesh = plsc.VectorSubcoreMesh(
        core_axis_name="c", subcore_axis_name="s",
        num_cores=NC, num_subcores=NS,
    )
    xs = jax.ShapeDtypeStruct((NC * NPAD, HALF), jnp.float32)
    return pl.kernel(
        _sc_body,
        out_type=[xs, xs, xs, xs, xs],
        mesh=mesh,
        scratch_types=[
            pltpu.VMEM_SHARED((NPAD, HALF), jnp.float32),
            pltpu.VMEM_SHARED((NPAD,), jnp.float32),
            pltpu.VMEM((SUPER, CHUNK), jnp.int32),
            pltpu.VMEM((SUPER, CHUNK), jnp.int32),
            pltpu.VMEM((STRIPE,), jnp.float32),
            pltpu.VMEM((CHUNK,), jnp.float32),
            pltpu.VMEM((16, HALF), jnp.float32),
            pltpu.VMEM((STRIPE,), jnp.float32),
            pltpu.VMEM((CHUNK, HALF), jnp.float32),
            pltpu.VMEM((16, HALF), jnp.float32),
            pltpu.VMEM((16, HALF), jnp.float32),
            pltpu.VMEM((16, HALF), jnp.float32),
            pltpu.SemaphoreType.DMA,
        ],
    )(x0_flat, er2, ec3)


def _combine_body(x0r, x1r, x2r, x3r, x4r, o_ref):
    o_ref[...] = (
        x0r[0] + 0.5 * x1r[0] + (1.0 / 3.0) * x2r[0]
        + 0.25 * x3r[0] + 0.2 * x4r[0]
    )


def _combine(xs):
    spec = pl.BlockSpec((1, 1000, HALF), lambda c, i: (c, i, 0))
    return pl.pallas_call(
        _combine_body,
        grid=(2, 10),
        in_specs=[spec] * 5,
        out_specs=pl.BlockSpec((1000, HALF), lambda c, i: (i, c)),
        out_shape=jax.ShapeDtypeStruct((N_NODES, D), jnp.float32),
    )(*xs)


def kernel(user_feature, movie_feature, edge_index, Wu, bu, Wm, bm):
    feat = jnp.stack([user_feature, movie_feature])
    W = jnp.stack([Wu, Wm])
    b = jnp.stack([bu, bm]).reshape(2, 1, 256)
    x0 = _embed(feat, W, b).reshape(N_NODES, D)

    x0s = x0.reshape(N_NODES, NC, HALF).transpose(1, 0, 2)
    x0_flat = jnp.pad(x0s, ((0, 0), (0, NPAD - N_NODES), (0, 0))).reshape(
        NC * NPAD, HALF)

    row = edge_index[0].astype(jnp.int32)
    col = edge_index[1].astype(jnp.int32)
    er2 = jnp.stack([row, row + NPAD]).reshape(NC, NS, NSUPER, SUPER, CHUNK)
    ec3 = col.reshape(NS, NSUPER, SUPER, CHUNK)

    x1, x2, x3, x4, _y = _propagate(x0_flat, er2, ec3)
    halves = [a.reshape(NC, NPAD, HALF) for a in (x0_flat, x1, x2, x3, x4)]
    return _combine(halves)


# speedup vs baseline: 9.6048x; 1.4980x over previous
import jax
import jax.numpy as jnp
from jax import lax
from jax.experimental import pallas as pl
from jax.experimental.pallas import tpu as pltpu
from jax.experimental.pallas import tpu_sc as plsc

NUM_LAYERS = 4
N_NODES = 10000
NPAD = 10240
D = 256
HALF = 128
NS = 16
NC = 2
STRIPE = NPAD // NS
E_PER_SUB = 160000 // NS
CHUNK = 80
NCHUNK = E_PER_SUB // CHUNK
SUPER = 25
NSUPER = NCHUNK // SUPER


def _embed_body(f_ref, w_ref, b_ref, o_ref):
    o_ref[...] = (
        lax.dot_general(
            f_ref[0], w_ref[0], (((1,), (1,)), ((), ())),
            preferred_element_type=jnp.float32,
        )[None]
        + b_ref[...]
    )


def _embed(feat, W, b):
    return pl.pallas_call(
        _embed_body,
        grid=(2, 5),
        in_specs=[
            pl.BlockSpec((1, 1000, 256), lambda i, j: (i, j, 0)),
            pl.BlockSpec((1, 256, 256), lambda i, j: (i, 0, 0)),
            pl.BlockSpec((1, 1, 256), lambda i, j: (i, 0, 0)),
        ],
        out_specs=pl.BlockSpec((1, 1000, 256), lambda i, j: (i, j, 0)),
        out_shape=jax.ShapeDtypeStruct((2, 5000, 256), jnp.float32),
    )(feat, W, b)


def _sc_body(x0_hbm, er_hbm, ec_hbm,
             xk_hbm, y_hbm,
             acc_sh, deg_sh, rowi_v, coli_v, dinv_v, ones_v,
             zrow_v, zdeg_v, gath0_v, gath1_v, buf_a, buf_x, buf_y,
             gsem0, gsem1, ssem0, ssem1, zsem):
    c = lax.axis_index("c")
    s = lax.axis_index("s")
    my0 = s * STRIPE
    base = c * NPAD + my0
    zero16 = jnp.zeros((16,), jnp.float32)
    one16 = jnp.ones((16,), jnp.float32)
    for i in range(STRIPE // 16):
        zdeg_v[pl.ds(i * 16, 16)] = zero16
    for i in range(16):
        for j in range(HALF // 16):
            zrow_v[i, pl.ds(j * 16, 16)] = zero16
    for i in range(CHUNK // 16):
        ones_v[pl.ds(i * 16, 16)] = one16

    pltpu.sync_copy(zdeg_v, deg_sh.at[pl.ds(my0, STRIPE)])
    plsc.subcore_barrier()

    def deg_sup(g, carry):
        pltpu.sync_copy(ec_hbm.at[s, g], coli_v)
        descs = [
            pltpu.async_copy(ones_v, deg_sh.at[coli_v.at[j]], zsem, add=True)
            for j in range(SUPER)
        ]
        for d in descs:
            d.wait()
        return carry
    lax.fori_loop(0, NSUPER, deg_sup, 0)
    plsc.subcore_barrier()

    pltpu.sync_copy(deg_sh.at[pl.ds(my0, STRIPE)], dinv_v)
    for i in range(STRIPE // 16):
        d = dinv_v[pl.ds(i * 16, 16)]
        bits = lax.bitcast_convert_type(d, jnp.int32)
        y = lax.bitcast_convert_type(jnp.int32(0x5F3759DF) - (bits >> 1), jnp.float32)
        for _ in range(3):
            y = y * (1.5 - 0.5 * d * y * y)
        dinv_v[pl.ds(i * 16, 16)] = jnp.where(d > 0, y, 0.0)

    def y0_step(k, carry):
        r0 = base + k * 16
        pltpu.sync_copy(x0_hbm.at[pl.ds(r0, 16)], buf_x)
        dv = dinv_v[pl.ds(k * 16, 16)]
        for r in range(16):
            d = dv[r]
            for jj in range(HALF // 16):
                buf_y[r, pl.ds(jj * 16, 16)] = buf_x[r, pl.ds(jj * 16, 16)] * d
        pltpu.sync_copy(buf_y, y_hbm.at[pl.ds(r0, 16)])
        return carry
    lax.fori_loop(0, STRIPE // 16, y0_step, 0)

    gbufs = (gath0_v, gath1_v)
    gsems = (gsem0, gsem1)
    ssems = (ssem0, ssem1)

    def layer(ell, carry):
        zdescs = [
            pltpu.async_copy(zrow_v, acc_sh.at[pl.ds(my0 + k * 16, 16)], zsem)
            for k in range(STRIPE // 16)
        ]
        for d in zdescs:
            d.wait()
        plsc.subcore_barrier()

        def gs_sup(g, carry2):
            pltpu.sync_copy(er_hbm.at[c, s, g], rowi_v)
            pltpu.sync_copy(ec_hbm.at[s, g], coli_v)
            gd = [None, None]
            sd = [None, None]
            gd[0] = pltpu.async_copy(y_hbm.at[rowi_v.at[0]], gbufs[0],
                                     gsems[0])
            for j in range(SUPER):
                cur = j % 2
                oth = 1 - cur
                if j + 1 < SUPER:
                    if sd[oth] is not None:
                        sd[oth].wait()
                    gd[oth] = pltpu.async_copy(
                        y_hbm.at[rowi_v.at[j + 1]], gbufs[oth], gsems[oth])
                gd[cur].wait()
                sd[cur] = pltpu.async_copy(
                    gbufs[cur], acc_sh.at[coli_v.at[j]], ssems[cur], add=True)
            sd[0].wait()
            sd[1].wait()
            return carry2
        lax.fori_loop(0, NSUPER, gs_sup, 0)
        plsc.subcore_barrier()

        def o_step(k, carry2):
            pltpu.sync_copy(acc_sh.at[pl.ds(my0 + k * 16, 16)], buf_a)
            dv = dinv_v[pl.ds(k * 16, 16)]
            for r in range(16):
                d = dv[r]
                d2 = d * d
                for jj in range(HALF // 16):
                    a = buf_a[r, pl.ds(jj * 16, 16)]
                    buf_x[r, pl.ds(jj * 16, 16)] = a * d
                    buf_y[r, pl.ds(jj * 16, 16)] = a * d2
            r0 = base + k * 16
            pltpu.sync_copy(buf_x, xk_hbm.at[ell].at[pl.ds(r0, 16)])
            pltpu.sync_copy(buf_y, y_hbm.at[pl.ds(r0, 16)])
            return carry2
        lax.fori_loop(0, STRIPE // 16, o_step, 0)
        return carry

    lax.fori_loop(0, NUM_LAYERS, layer, 0)


def _propagate(x0_flat, er2, ec3):
    mesh = plsc.VectorSubcoreMesh(
        core_axis_name="c", subcore_axis_name="s",
        num_cores=NC, num_subcores=NS,
    )
    xk = jax.ShapeDtypeStruct((NUM_LAYERS, NC * NPAD, HALF), jnp.float32)
    ys = jax.ShapeDtypeStruct((NC * NPAD, HALF), jnp.float32)
    return pl.kernel(
        _sc_body,
        out_type=[xk, ys],
        mesh=mesh,
        scratch_types=[
            pltpu.VMEM_SHARED((NPAD, HALF), jnp.float32),
            pltpu.VMEM_SHARED((NPAD,), jnp.float32),
            pltpu.VMEM((SUPER, CHUNK), jnp.int32),
            pltpu.VMEM((SUPER, CHUNK), jnp.int32),
            pltpu.VMEM((STRIPE,), jnp.float32),
            pltpu.VMEM((CHUNK,), jnp.float32),
            pltpu.VMEM((16, HALF), jnp.float32),
            pltpu.VMEM((STRIPE,), jnp.float32),
            pltpu.VMEM((CHUNK, HALF), jnp.float32),
            pltpu.VMEM((CHUNK, HALF), jnp.float32),
            pltpu.VMEM((16, HALF), jnp.float32),
            pltpu.VMEM((16, HALF), jnp.float32),
            pltpu.VMEM((16, HALF), jnp.float32),
            pltpu.SemaphoreType.DMA,
            pltpu.SemaphoreType.DMA,
            pltpu.SemaphoreType.DMA,
            pltpu.SemaphoreType.DMA,
            pltpu.SemaphoreType.DMA,
        ],
    )(x0_flat, er2, ec3)


def _combine_body(x0r, xkr, o_ref):
    o_ref[...] = (
        x0r[0] + 0.5 * xkr[0, 0] + (1.0 / 3.0) * xkr[1, 0]
        + 0.25 * xkr[2, 0] + 0.2 * xkr[3, 0]
    )


def _combine(x0s, xk):
    return pl.pallas_call(
        _combine_body,
        grid=(2, 10),
        in_specs=[
            pl.BlockSpec((1, 1000, HALF), lambda c, i: (c, i, 0)),
            pl.BlockSpec((NUM_LAYERS, 1, 1000, HALF),
                         lambda c, i: (0, c, i, 0)),
        ],
        out_specs=pl.BlockSpec((1000, HALF), lambda c, i: (i, c)),
        out_shape=jax.ShapeDtypeStruct((N_NODES, D), jnp.float32),
    )(x0s, xk)


def kernel(user_feature, movie_feature, edge_index, Wu, bu, Wm, bm):
    feat = jnp.stack([user_feature, movie_feature])
    W = jnp.stack([Wu, Wm])
    b = jnp.stack([bu, bm]).reshape(2, 1, 256)
    x0 = _embed(feat, W, b).reshape(N_NODES, D)

    x0s = x0.reshape(N_NODES, NC, HALF).transpose(1, 0, 2)
    x0_flat = jnp.pad(x0s, ((0, 0), (0, NPAD - N_NODES), (0, 0))).reshape(
        NC * NPAD, HALF)

    row = edge_index[0].astype(jnp.int32)
    col = edge_index[1].astype(jnp.int32)
    er2 = jnp.stack([row, row + NPAD]).reshape(NC, NS, NSUPER, SUPER, CHUNK)
    ec3 = col.reshape(NS, NSUPER, SUPER, CHUNK)

    xk, _y = _propagate(x0_flat, er2, ec3)
    return _combine(x0_flat.reshape(NC, NPAD, HALF),
                    xk.reshape(NUM_LAYERS, NC, NPAD, HALF))


# speedup vs baseline: 10.6580x; 1.1096x over previous
import jax
import jax.numpy as jnp
from jax import lax
from jax.experimental import pallas as pl
from jax.experimental.pallas import tpu as pltpu
from jax.experimental.pallas import tpu_sc as plsc

NUM_LAYERS = 4
N_NODES = 10000
NPAD = 10240
D = 256
HALF = 128
NS = 16
NC = 2
STRIPE = NPAD // NS
NB = STRIPE // 16
E_PER_SUB = 160000 // NS
CHUNK = 80
NCHUNK = E_PER_SUB // CHUNK
SUPER = 25
NSUPER = NCHUNK // SUPER


def _embed_body(f_ref, w_ref, b_ref, o_ref):
    o_ref[...] = (
        lax.dot_general(
            f_ref[0], w_ref[0], (((1,), (1,)), ((), ())),
            preferred_element_type=jnp.float32,
        )[None]
        + b_ref[...]
    )


def _embed(feat, W, b):
    return pl.pallas_call(
        _embed_body,
        grid=(2, 5),
        in_specs=[
            pl.BlockSpec((1, 1000, 256), lambda i, j: (i, j, 0)),
            pl.BlockSpec((1, 256, 256), lambda i, j: (i, 0, 0)),
            pl.BlockSpec((1, 1, 256), lambda i, j: (i, 0, 0)),
        ],
        out_specs=pl.BlockSpec((1, 1000, 256), lambda i, j: (i, j, 0)),
        out_shape=jax.ShapeDtypeStruct((2, 5000, 256), jnp.float32),
    )(feat, W, b)


def _sc_body(x0_hbm, er_hbm, ec_hbm,
             yk_hbm, degs_hbm,
             acc_sh, deg_sh, rowi_v, coli_v, dinv_v, ones_v,
             zrow_v, zdeg_v, gath0_v, gath1_v, bufa0_v, bufa1_v, buf_y,
             gsem0, gsem1, ssem0, ssem1, zsem, psem, asem0, asem1):
    c = lax.axis_index("c")
    s = lax.axis_index("s")
    my0 = s * STRIPE
    base = c * NPAD + my0
    zero16 = jnp.zeros((16,), jnp.float32)
    one16 = jnp.ones((16,), jnp.float32)
    for i in range(STRIPE // 16):
        zdeg_v[pl.ds(i * 16, 16)] = zero16
    for i in range(16):
        for j in range(HALF // 16):
            zrow_v[i, pl.ds(j * 16, 16)] = zero16
    for i in range(CHUNK // 16):
        ones_v[pl.ds(i * 16, 16)] = one16

    abufs = (bufa0_v, bufa1_v)
    asems = (asem0, asem1)

    def scaled_pass(read_block, write_ref_at, square, rezero):
        def issue(k, b):
            return pltpu.async_copy(read_block(k), abufs[b], asems[b])

        issue(0, 0)
        issue(1, 1)

        def blk(k, b):
            pltpu.make_async_copy(read_block(k), abufs[b], asems[b]).wait()
            if rezero:
                pltpu.async_copy(
                    zrow_v, acc_sh.at[pl.ds(my0 + k * 16, 16)], zsem)
            dv = dinv_v[pl.ds(k * 16, 16)]
            buf = abufs[b]
            for r in range(16):
                d = dv[r]
                if square:
                    d = d * d
                for jj in range(HALF // 16):
                    buf_y[r, pl.ds(jj * 16, 16)] = (
                        buf[r, pl.ds(jj * 16, 16)] * d)
            pltpu.sync_copy(buf_y, write_ref_at(k))

        def body(kk, carry):
            for b in range(2):
                k = 2 * kk + b
                blk(k, b)

                @pl.when(kk < NB // 2 - 1)
                def _():
                    issue(k + 2, b)
            return carry
        lax.fori_loop(0, NB // 2, body, 0)
        if rezero:
            def drain(k, carry):
                pltpu.make_async_copy(
                    zrow_v, acc_sh.at[pl.ds(my0, 16)], zsem).wait()
                return carry
            lax.fori_loop(0, NB, drain, 0)

    for k in range(NB):
        pltpu.async_copy(zrow_v, acc_sh.at[pl.ds(my0 + k * 16, 16)], psem)
    pltpu.sync_copy(zdeg_v, deg_sh.at[pl.ds(my0, STRIPE)])
    plsc.subcore_barrier()

    def deg_sup(g, carry):
        pltpu.sync_copy(ec_hbm.at[s, g], coli_v)
        descs = [
            pltpu.async_copy(ones_v, deg_sh.at[coli_v.at[j]], zsem, add=True)
            for j in range(SUPER)
        ]
        for d in descs:
            d.wait()
        return carry
    lax.fori_loop(0, NSUPER, deg_sup, 0)
    plsc.subcore_barrier()

    pltpu.sync_copy(deg_sh.at[pl.ds(my0, STRIPE)], dinv_v)
    pltpu.sync_copy(dinv_v, degs_hbm.at[c, pl.ds(my0, STRIPE)])

    def newton(i, carry):
        d = dinv_v[pl.ds(i * 16, 16)]
        bits = lax.bitcast_convert_type(d, jnp.int32)
        y = lax.bitcast_convert_type(
            jnp.int32(0x5F3759DF) - (bits >> 1), jnp.float32)
        for _ in range(3):
            y = y * (1.5 - 0.5 * d * y * y)
        dinv_v[pl.ds(i * 16, 16)] = jnp.where(d > 0, y, 0.0)
        return carry
    lax.fori_loop(0, STRIPE // 16, newton, 0)

    scaled_pass(
        lambda k: x0_hbm.at[pl.ds(base + k * 16, 16)],
        lambda k: yk_hbm.at[0, pl.ds(base + k * 16, 16)],
        square=False, rezero=False,
    )
    def pdrain(k, carry):
        pltpu.make_async_copy(
            zrow_v, acc_sh.at[pl.ds(my0, 16)], psem).wait()
        return carry
    lax.fori_loop(0, NB, pdrain, 0)
    plsc.subcore_barrier()

    gbufs = (gath0_v, gath1_v)
    gsems = (gsem0, gsem1)
    ssems = (ssem0, ssem1)

    def layer(ell, carry):
        def gs_sup(g, carry2):
            pltpu.sync_copy(er_hbm.at[c, s, g], rowi_v)
            pltpu.sync_copy(ec_hbm.at[s, g], coli_v)
            src = yk_hbm.at[ell]
            gd = [None, None]
            sd = [None, None]
            gd[0] = pltpu.async_copy(src.at[rowi_v.at[0]], gbufs[0],
                                     gsems[0])
            for j in range(SUPER):
                cur = j % 2
                oth = 1 - cur
                if j + 1 < SUPER:
                    if sd[oth] is not None:
                        sd[oth].wait()
                    gd[oth] = pltpu.async_copy(
                        src.at[rowi_v.at[j + 1]], gbufs[oth], gsems[oth])
                gd[cur].wait()
                sd[cur] = pltpu.async_copy(
                    gbufs[cur], acc_sh.at[coli_v.at[j]], ssems[cur], add=True)
            sd[0].wait()
            sd[1].wait()
            return carry2
        lax.fori_loop(0, NSUPER, gs_sup, 0)
        plsc.subcore_barrier()

        scaled_pass(
            lambda k: acc_sh.at[pl.ds(my0 + k * 16, 16)],
            lambda k: yk_hbm.at[ell + 1, pl.ds(base + k * 16, 16)],
            square=True, rezero=True,
        )
        plsc.subcore_barrier()
        return carry

    lax.fori_loop(0, NUM_LAYERS, layer, 0)


def _propagate(x0_flat, er2, ec3):
    mesh = plsc.VectorSubcoreMesh(
        core_axis_name="c", subcore_axis_name="s",
        num_cores=NC, num_subcores=NS,
    )
    return pl.kernel(
        _sc_body,
        out_type=[
            jax.ShapeDtypeStruct((NUM_LAYERS + 1, NC * NPAD, HALF),
                                 jnp.float32),
            jax.ShapeDtypeStruct((NC, NPAD), jnp.float32),
        ],
        mesh=mesh,
        scratch_types=[
            pltpu.VMEM_SHARED((NPAD, HALF), jnp.float32),
            pltpu.VMEM_SHARED((NPAD,), jnp.float32),
            pltpu.VMEM((SUPER, CHUNK), jnp.int32),
            pltpu.VMEM((SUPER, CHUNK), jnp.int32),
            pltpu.VMEM((STRIPE,), jnp.float32),
            pltpu.VMEM((CHUNK,), jnp.float32),
            pltpu.VMEM((16, HALF), jnp.float32),
            pltpu.VMEM((STRIPE,), jnp.float32),
            pltpu.VMEM((CHUNK, HALF), jnp.float32),
            pltpu.VMEM((CHUNK, HALF), jnp.float32),
            pltpu.VMEM((16, HALF), jnp.float32),
            pltpu.VMEM((16, HALF), jnp.float32),
            pltpu.VMEM((16, HALF), jnp.float32),
            pltpu.SemaphoreType.DMA,
            pltpu.SemaphoreType.DMA,
            pltpu.SemaphoreType.DMA,
            pltpu.SemaphoreType.DMA,
            pltpu.SemaphoreType.DMA,
            pltpu.SemaphoreType.DMA,
            pltpu.SemaphoreType.DMA,
            pltpu.SemaphoreType.DMA,
        ],
    )(x0_flat, er2, ec3)


def _combine_body(x0r, ykr, degr, o_ref):
    sq = jnp.sqrt(degr[0])
    o_ref[...] = x0r[0] + sq * (
        0.5 * ykr[1, 0] + (1.0 / 3.0) * ykr[2, 0]
        + 0.25 * ykr[3, 0] + 0.2 * ykr[4, 0]
    )


def _combine(x0s, yk, degs):
    return pl.pallas_call(
        _combine_body,
        grid=(2, 10),
        in_specs=[
            pl.BlockSpec((1, 1000, HALF), lambda c, i: (c, i, 0)),
            pl.BlockSpec((NUM_LAYERS + 1, 1, 1000, HALF),
                         lambda c, i: (0, c, i, 0)),
            pl.BlockSpec((1, 1000, 1), lambda c, i: (c, i, 0)),
        ],
        out_specs=pl.BlockSpec((1000, HALF), lambda c, i: (i, c)),
        out_shape=jax.ShapeDtypeStruct((N_NODES, D), jnp.float32),
    )(x0s, yk, degs)


def kernel(user_feature, movie_feature, edge_index, Wu, bu, Wm, bm):
    feat = jnp.stack([user_feature, movie_feature])
    W = jnp.stack([Wu, Wm])
    b = jnp.stack([bu, bm]).reshape(2, 1, 256)
    x0 = _embed(feat, W, b).reshape(N_NODES, D)

    x0s = x0.reshape(N_NODES, NC, HALF).transpose(1, 0, 2)
    x0_flat = jnp.pad(x0s, ((0, 0), (0, NPAD - N_NODES), (0, 0))).reshape(
        NC * NPAD, HALF)

    row = edge_index[0].astype(jnp.int32)
    col = edge_index[1].astype(jnp.int32)
    er2 = jnp.stack([row, row + NPAD]).reshape(NC, NS, NSUPER, SUPER, CHUNK)
    ec3 = col.reshape(NS, NSUPER, SUPER, CHUNK)

    yk, degs = _propagate(x0_flat, er2, ec3)
    return _combine(
        x0_flat.reshape(NC, NPAD, HALF),
        yk.reshape(NUM_LAYERS + 1, NC, NPAD, HALF),
        degs.reshape(NC, NPAD, 1),
    )


# speedup vs baseline: 11.3172x; 1.0619x over previous
import jax
import jax.numpy as jnp
from jax import lax
from jax.experimental import pallas as pl
from jax.experimental.pallas import tpu as pltpu
from jax.experimental.pallas import tpu_sc as plsc

NUM_LAYERS = 4
N_NODES = 10000
NPAD = 10240
D = 256
HALF = 128
NS = 16
NC = 2
STRIPE = NPAD // NS
NB = STRIPE // 16
E_PER_SUB = 10240
CHUNK = 128
NCHUNK = E_PER_SUB // CHUNK
SUPER = 16
NSUPER = NCHUNK // SUPER


def _embed_body(f_ref, w_ref, b_ref, o_ref):
    o_ref[...] = (
        lax.dot_general(
            f_ref[0], w_ref[0], (((1,), (1,)), ((), ())),
            preferred_element_type=jnp.float32,
        )[None]
        + b_ref[...]
    )


def _embed(feat, W, b):
    return pl.pallas_call(
        _embed_body,
        grid=(2, 5),
        in_specs=[
            pl.BlockSpec((1, 1000, 256), lambda i, j: (i, j, 0)),
            pl.BlockSpec((1, 256, 256), lambda i, j: (i, 0, 0)),
            pl.BlockSpec((1, 1, 256), lambda i, j: (i, 0, 0)),
        ],
        out_specs=pl.BlockSpec((1, 1000, 256), lambda i, j: (i, j, 0)),
        out_shape=jax.ShapeDtypeStruct((2, 5000, 256), jnp.float32),
    )(feat, W, b)


def _sc_body(x0_hbm, er_hbm, ec_hbm,
             yk_hbm, degs_hbm,
             acc_sh, deg_sh, rowi_v, coli_v, dinv_v, ones_v,
             zrow_v, zdeg_v, gath0_v, gath1_v, bufa0_v, bufa1_v, buf_y,
             gsem0, gsem1, ssem0, ssem1, zsem, psem, asem0, asem1):
    c = lax.axis_index("c")
    s = lax.axis_index("s")
    my0 = s * STRIPE
    base = c * NPAD + my0
    zero16 = jnp.zeros((16,), jnp.float32)
    one16 = jnp.ones((16,), jnp.float32)
    for i in range(STRIPE // 16):
        zdeg_v[pl.ds(i * 16, 16)] = zero16
    for i in range(16):
        for j in range(HALF // 16):
            zrow_v[i, pl.ds(j * 16, 16)] = zero16
    for i in range(CHUNK // 16):
        ones_v[pl.ds(i * 16, 16)] = one16

    abufs = (bufa0_v, bufa1_v)
    asems = (asem0, asem1)

    def scaled_pass(read_block, write_ref_at, square, rezero):
        def issue(k, b):
            return pltpu.async_copy(read_block(k), abufs[b], asems[b])

        issue(0, 0)
        issue(1, 1)

        def blk(k, b):
            pltpu.make_async_copy(read_block(k), abufs[b], asems[b]).wait()
            if rezero:
                pltpu.async_copy(
                    zrow_v, acc_sh.at[pl.ds(my0 + k * 16, 16)], zsem)
            dv = dinv_v[pl.ds(k * 16, 16)]
            buf = abufs[b]
            for r in range(16):
                d = dv[r]
                if square:
                    d = d * d
                for jj in range(HALF // 16):
                    buf_y[r, pl.ds(jj * 16, 16)] = (
                        buf[r, pl.ds(jj * 16, 16)] * d)
            pltpu.sync_copy(buf_y, write_ref_at(k))

        def body(kk, carry):
            for b in range(2):
                k = 2 * kk + b
                blk(k, b)

                @pl.when(kk < NB // 2 - 1)
                def _():
                    issue(k + 2, b)
            return carry
        lax.fori_loop(0, NB // 2, body, 0)
        if rezero:
            def drain(k, carry):
                pltpu.make_async_copy(
                    zrow_v, acc_sh.at[pl.ds(my0, 16)], zsem).wait()
                return carry
            lax.fori_loop(0, NB, drain, 0)

    for k in range(NB):
        pltpu.async_copy(zrow_v, acc_sh.at[pl.ds(my0 + k * 16, 16)], psem)
    pltpu.sync_copy(zdeg_v, deg_sh.at[pl.ds(my0, STRIPE)])
    plsc.subcore_barrier()

    def deg_sup(g, carry):
        pltpu.sync_copy(ec_hbm.at[s, g], coli_v)
        descs = [
            pltpu.async_copy(ones_v, deg_sh.at[coli_v.at[j]], zsem, add=True)
            for j in range(SUPER)
        ]
        for d in descs:
            d.wait()
        return carry
    lax.fori_loop(0, NSUPER, deg_sup, 0)
    plsc.subcore_barrier()

    pltpu.sync_copy(deg_sh.at[pl.ds(my0, STRIPE)], dinv_v)
    pltpu.sync_copy(dinv_v, degs_hbm.at[c, pl.ds(my0, STRIPE)])

    def newton(i, carry):
        d = dinv_v[pl.ds(i * 16, 16)]
        bits = lax.bitcast_convert_type(d, jnp.int32)
        y = lax.bitcast_convert_type(
            jnp.int32(0x5F3759DF) - (bits >> 1), jnp.float32)
        for _ in range(3):
            y = y * (1.5 - 0.5 * d * y * y)
        dinv_v[pl.ds(i * 16, 16)] = jnp.where(d > 0, y, 0.0)
        return carry
    lax.fori_loop(0, STRIPE // 16, newton, 0)

    scaled_pass(
        lambda k: x0_hbm.at[pl.ds(base + k * 16, 16)],
        lambda k: yk_hbm.at[0, pl.ds(base + k * 16, 16)],
        square=False, rezero=False,
    )
    def pdrain(k, carry):
        pltpu.make_async_copy(
            zrow_v, acc_sh.at[pl.ds(my0, 16)], psem).wait()
        return carry
    lax.fori_loop(0, NB, pdrain, 0)
    plsc.subcore_barrier()

    gbufs = (gath0_v, gath1_v)
    gsems = (gsem0, gsem1)
    ssems = (ssem0, ssem1)

    def layer(ell, carry):
        def gs_sup(g, carry2):
            pltpu.sync_copy(er_hbm.at[c, s, g], rowi_v)
            pltpu.sync_copy(ec_hbm.at[s, g], coli_v)
            src = yk_hbm.at[ell]
            gd = [None, None]
            sd = [None, None]
            gd[0] = pltpu.async_copy(src.at[rowi_v.at[0]], gbufs[0],
                                     gsems[0])
            for j in range(SUPER):
                cur = j % 2
                oth = 1 - cur
                if j + 1 < SUPER:
                    if sd[oth] is not None:
                        sd[oth].wait()
                    gd[oth] = pltpu.async_copy(
                        src.at[rowi_v.at[j + 1]], gbufs[oth], gsems[oth])
                gd[cur].wait()
                sd[cur] = pltpu.async_copy(
                    gbufs[cur], acc_sh.at[coli_v.at[j]], ssems[cur], add=True)
            sd[0].wait()
            sd[1].wait()
            return carry2
        lax.fori_loop(0, NSUPER, gs_sup, 0)
        plsc.subcore_barrier()

        scaled_pass(
            lambda k: acc_sh.at[pl.ds(my0 + k * 16, 16)],
            lambda k: yk_hbm.at[ell + 1, pl.ds(base + k * 16, 16)],
            square=True, rezero=True,
        )
        plsc.subcore_barrier()
        return carry

    lax.fori_loop(0, NUM_LAYERS, layer, 0)


def _propagate(x0_flat, er2, ec3):
    mesh = plsc.VectorSubcoreMesh(
        core_axis_name="c", subcore_axis_name="s",
        num_cores=NC, num_subcores=NS,
    )
    return pl.kernel(
        _sc_body,
        out_type=[
            jax.ShapeDtypeStruct((NUM_LAYERS + 1, NC * NPAD, HALF),
                                 jnp.float32),
            jax.ShapeDtypeStruct((NC, NPAD), jnp.float32),
        ],
        mesh=mesh,
        scratch_types=[
            pltpu.VMEM_SHARED((NPAD, HALF), jnp.float32),
            pltpu.VMEM_SHARED((NPAD,), jnp.float32),
            pltpu.VMEM((SUPER, CHUNK), jnp.int32),
            pltpu.VMEM((SUPER, CHUNK), jnp.int32),
            pltpu.VMEM((STRIPE,), jnp.float32),
            pltpu.VMEM((CHUNK,), jnp.float32),
            pltpu.VMEM((16, HALF), jnp.float32),
            pltpu.VMEM((STRIPE,), jnp.float32),
            pltpu.VMEM((CHUNK, HALF), jnp.float32),
            pltpu.VMEM((CHUNK, HALF), jnp.float32),
            pltpu.VMEM((16, HALF), jnp.float32),
            pltpu.VMEM((16, HALF), jnp.float32),
            pltpu.VMEM((16, HALF), jnp.float32),
            pltpu.SemaphoreType.DMA,
            pltpu.SemaphoreType.DMA,
            pltpu.SemaphoreType.DMA,
            pltpu.SemaphoreType.DMA,
            pltpu.SemaphoreType.DMA,
            pltpu.SemaphoreType.DMA,
            pltpu.SemaphoreType.DMA,
            pltpu.SemaphoreType.DMA,
        ],
    )(x0_flat, er2, ec3)


def _combine_body(x0r, ykr, degr, o_ref):
    sq = jnp.sqrt(degr[0])
    o_ref[...] = x0r[0] + sq * (
        0.5 * ykr[1, 0] + (1.0 / 3.0) * ykr[2, 0]
        + 0.25 * ykr[3, 0] + 0.2 * ykr[4, 0]
    )


def _combine(x0s, yk, degs):
    return pl.pallas_call(
        _combine_body,
        grid=(2, 10),
        in_specs=[
            pl.BlockSpec((1, 1000, HALF), lambda c, i: (c, i, 0)),
            pl.BlockSpec((NUM_LAYERS + 1, 1, 1000, HALF),
                         lambda c, i: (0, c, i, 0)),
            pl.BlockSpec((1, 1000, 1), lambda c, i: (c, i, 0)),
        ],
        out_specs=pl.BlockSpec((1000, HALF), lambda c, i: (i, c)),
        out_shape=jax.ShapeDtypeStruct((N_NODES, D), jnp.float32),
    )(x0s, yk, degs)


def kernel(user_feature, movie_feature, edge_index, Wu, bu, Wm, bm):
    feat = jnp.stack([user_feature, movie_feature])
    W = jnp.stack([Wu, Wm])
    b = jnp.stack([bu, bm]).reshape(2, 1, 256)
    x0 = _embed(feat, W, b).reshape(N_NODES, D)

    x0s = x0.reshape(N_NODES, NC, HALF).transpose(1, 0, 2)
    x0_flat = jnp.pad(x0s, ((0, 0), (0, NPAD - N_NODES), (0, 0))).reshape(
        NC * NPAD, HALF)

    row = edge_index[0].astype(jnp.int32)
    col = edge_index[1].astype(jnp.int32)
    pad_idx = N_NODES + (jnp.arange(NS * E_PER_SUB - row.shape[0],
                                    dtype=jnp.int32) % (NPAD - N_NODES))
    row = jnp.concatenate([row, pad_idx])
    col = jnp.concatenate([col, pad_idx])
    er2 = jnp.stack([row, row + NPAD]).reshape(NC, NS, NSUPER, SUPER, CHUNK)
    ec3 = col.reshape(NS, NSUPER, SUPER, CHUNK)

    yk, degs = _propagate(x0_flat, er2, ec3)
    return _combine(
        x0_flat.reshape(NC, NPAD, HALF),
        yk.reshape(NUM_LAYERS + 1, NC, NPAD, HALF),
        degs.reshape(NC, NPAD, 1),
    )


# speedup vs baseline: 11.6083x; 1.0257x over previous
import jax
import jax.numpy as jnp
from jax import lax
from jax.experimental import pallas as pl
from jax.experimental.pallas import tpu as pltpu
from jax.experimental.pallas import tpu_sc as plsc

NUM_LAYERS = 4
N_NODES = 10000
NPAD = 10240
D = 256
HALF = 128
NS = 16
NC = 2
STRIPE = NPAD // NS
NB = STRIPE // 16
E_PER_SUB = 10240
CHUNK = 128
NCHUNK = E_PER_SUB // CHUNK
SUPER = 16
NSUPER = NCHUNK // SUPER


def _embed_body(f_ref, w_ref, b_ref, o_ref):
    o_ref[...] = (
        lax.dot_general(
            f_ref[0], w_ref[0, 0], (((1,), (1,)), ((), ())),
            preferred_element_type=jnp.float32,
        )
        + b_ref[0]
    )


def _embed(feat, W, b):
    return pl.pallas_call(
        _embed_body,
        grid=(2, 2, 5),
        in_specs=[
            pl.BlockSpec((1, 1000, 256), lambda t, c, j: (t, j, 0)),
            pl.BlockSpec((1, 1, 128, 256), lambda t, c, j: (t, c, 0, 0)),
            pl.BlockSpec((1, 1, 1, 128), lambda t, c, j: (t, c, 0, 0)),
        ],
        out_specs=pl.BlockSpec((1, 1000, 128),
                               lambda t, c, j: (c, 5 * t + j, 0)),
        out_shape=jax.ShapeDtypeStruct((NC, NPAD, HALF), jnp.float32),
    )(feat, W, b)


def _sc_body(x0_hbm, er_hbm, ec_hbm,
             yk_hbm, degs_hbm,
             acc_sh, deg_sh, rowi_v, coli_v, dinv_v, ones_v,
             zrow_v, zdeg_v, gath0_v, gath1_v, bufa0_v, bufa1_v, buf_y,
             gsem0, gsem1, ssem0, ssem1, zsem, psem, asem0, asem1):
    c = lax.axis_index("c")
    s = lax.axis_index("s")
    my0 = s * STRIPE
    base = c * NPAD + my0
    zero16 = jnp.zeros((16,), jnp.float32)
    one16 = jnp.ones((16,), jnp.float32)
    for i in range(STRIPE // 16):
        zdeg_v[pl.ds(i * 16, 16)] = zero16
    for i in range(16):
        for j in range(HALF // 16):
            zrow_v[i, pl.ds(j * 16, 16)] = zero16
    for i in range(CHUNK // 16):
        ones_v[pl.ds(i * 16, 16)] = one16

    abufs = (bufa0_v, bufa1_v)
    asems = (asem0, asem1)

    def scaled_pass(read_block, write_ref_at, square, rezero):
        def issue(k, b):
            return pltpu.async_copy(read_block(k), abufs[b], asems[b])

        issue(0, 0)
        issue(1, 1)

        def blk(k, b):
            pltpu.make_async_copy(read_block(k), abufs[b], asems[b]).wait()
            if rezero:
                pltpu.async_copy(
                    zrow_v, acc_sh.at[pl.ds(my0 + k * 16, 16)], zsem)
            dv = dinv_v[pl.ds(k * 16, 16)]
            buf = abufs[b]
            for r in range(16):
                d = dv[r]
                if square:
                    d = d * d
                for jj in range(HALF // 16):
                    buf_y[r, pl.ds(jj * 16, 16)] = (
                        buf[r, pl.ds(jj * 16, 16)] * d)
            pltpu.sync_copy(buf_y, write_ref_at(k))

        def body(kk, carry):
            for b in range(2):
                k = 2 * kk + b
                blk(k, b)

                @pl.when(kk < NB // 2 - 1)
                def _():
                    issue(k + 2, b)
            return carry
        lax.fori_loop(0, NB // 2, body, 0)
        if rezero:
            def drain(k, carry):
                pltpu.make_async_copy(
                    zrow_v, acc_sh.at[pl.ds(my0, 16)], zsem).wait()
                return carry
            lax.fori_loop(0, NB, drain, 0)

    for k in range(NB):
        pltpu.async_copy(zrow_v, acc_sh.at[pl.ds(my0 + k * 16, 16)], psem)
    pltpu.sync_copy(zdeg_v, deg_sh.at[pl.ds(my0, STRIPE)])
    plsc.subcore_barrier()

    def deg_sup(g, carry):
        pltpu.sync_copy(ec_hbm.at[s, g], coli_v)
        descs = [
            pltpu.async_copy(ones_v, deg_sh.at[coli_v.at[j]], zsem, add=True)
            for j in range(SUPER)
        ]
        for d in descs:
            d.wait()
        return carry
    lax.fori_loop(0, NSUPER, deg_sup, 0)
    plsc.subcore_barrier()

    pltpu.sync_copy(deg_sh.at[pl.ds(my0, STRIPE)], dinv_v)
    pltpu.sync_copy(dinv_v, degs_hbm.at[c, pl.ds(my0, STRIPE)])

    def newton(i, carry):
        d = dinv_v[pl.ds(i * 16, 16)]
        bits = lax.bitcast_convert_type(d, jnp.int32)
        y = lax.bitcast_convert_type(
            jnp.int32(0x5F3759DF) - (bits >> 1), jnp.float32)
        for _ in range(3):
            y = y * (1.5 - 0.5 * d * y * y)
        dinv_v[pl.ds(i * 16, 16)] = jnp.where(d > 0, y, 0.0)
        return carry
    lax.fori_loop(0, STRIPE // 16, newton, 0)

    scaled_pass(
        lambda k: x0_hbm.at[pl.ds(base + k * 16, 16)],
        lambda k: yk_hbm.at[0, pl.ds(base + k * 16, 16)],
        square=False, rezero=False,
    )
    def pdrain(k, carry):
        pltpu.make_async_copy(
            zrow_v, acc_sh.at[pl.ds(my0, 16)], psem).wait()
        return carry
    lax.fori_loop(0, NB, pdrain, 0)
    plsc.subcore_barrier()

    gbufs = (gath0_v, gath1_v)
    gsems = (gsem0, gsem1)
    ssems = (ssem0, ssem1)

    def layer(ell, carry):
        def gs_sup(g, carry2):
            pltpu.sync_copy(er_hbm.at[c, s, g], rowi_v)
            pltpu.sync_copy(ec_hbm.at[s, g], coli_v)
            src = yk_hbm.at[ell]
            gd = [None, None]
            sd = [None, None]
            gd[0] = pltpu.async_copy(src.at[rowi_v.at[0]], gbufs[0],
                                     gsems[0])
            for j in range(SUPER):
                cur = j % 2
                oth = 1 - cur
                if j + 1 < SUPER:
                    if sd[oth] is not None:
                        sd[oth].wait()
                    gd[oth] = pltpu.async_copy(
                        src.at[rowi_v.at[j + 1]], gbufs[oth], gsems[oth])
                gd[cur].wait()
                sd[cur] = pltpu.async_copy(
                    gbufs[cur], acc_sh.at[coli_v.at[j]], ssems[cur], add=True)
            sd[0].wait()
            sd[1].wait()
            return carry2
        lax.fori_loop(0, NSUPER, gs_sup, 0)
        plsc.subcore_barrier()

        scaled_pass(
            lambda k: acc_sh.at[pl.ds(my0 + k * 16, 16)],
            lambda k: yk_hbm.at[ell + 1, pl.ds(base + k * 16, 16)],
            square=True, rezero=True,
        )
        plsc.subcore_barrier()
        return carry

    lax.fori_loop(0, NUM_LAYERS, layer, 0)


def _propagate(x0_flat, er2, ec3):
    mesh = plsc.VectorSubcoreMesh(
        core_axis_name="c", subcore_axis_name="s",
        num_cores=NC, num_subcores=NS,
    )
    return pl.kernel(
        _sc_body,
        out_type=[
            jax.ShapeDtypeStruct((NUM_LAYERS + 1, NC * NPAD, HALF),
                                 jnp.float32),
            jax.ShapeDtypeStruct((NC, NPAD), jnp.float32),
        ],
        mesh=mesh,
        scratch_types=[
            pltpu.VMEM_SHARED((NPAD, HALF), jnp.float32),
            pltpu.VMEM_SHARED((NPAD,), jnp.float32),
            pltpu.VMEM((SUPER, CHUNK), jnp.int32),
            pltpu.VMEM((SUPER, CHUNK), jnp.int32),
            pltpu.VMEM((STRIPE,), jnp.float32),
            pltpu.VMEM((CHUNK,), jnp.float32),
            pltpu.VMEM((16, HALF), jnp.float32),
            pltpu.VMEM((STRIPE,), jnp.float32),
            pltpu.VMEM((CHUNK, HALF), jnp.float32),
            pltpu.VMEM((CHUNK, HALF), jnp.float32),
            pltpu.VMEM((16, HALF), jnp.float32),
            pltpu.VMEM((16, HALF), jnp.float32),
            pltpu.VMEM((16, HALF), jnp.float32),
            pltpu.SemaphoreType.DMA,
            pltpu.SemaphoreType.DMA,
            pltpu.SemaphoreType.DMA,
            pltpu.SemaphoreType.DMA,
            pltpu.SemaphoreType.DMA,
            pltpu.SemaphoreType.DMA,
            pltpu.SemaphoreType.DMA,
            pltpu.SemaphoreType.DMA,
        ],
    )(x0_flat, er2, ec3)


def _combine_body(x0r, ykr, degr, o_ref):
    sq = jnp.sqrt(degr[0])
    o_ref[...] = x0r[0] + sq * (
        0.5 * ykr[1, 0] + (1.0 / 3.0) * ykr[2, 0]
        + 0.25 * ykr[3, 0] + 0.2 * ykr[4, 0]
    )


def _combine(x0s, yk, degs):
    return pl.pallas_call(
        _combine_body,
        grid=(2, 10),
        in_specs=[
            pl.BlockSpec((1, 1000, HALF), lambda c, i: (c, i, 0)),
            pl.BlockSpec((NUM_LAYERS + 1, 1, 1000, HALF),
                         lambda c, i: (0, c, i, 0)),
            pl.BlockSpec((1, 1000, 1), lambda c, i: (c, i, 0)),
        ],
        out_specs=pl.BlockSpec((1000, HALF), lambda c, i: (i, c)),
        out_shape=jax.ShapeDtypeStruct((N_NODES, D), jnp.float32),
    )(x0s, yk, degs)


def kernel(user_feature, movie_feature, edge_index, Wu, bu, Wm, bm):
    feat = jnp.stack([user_feature, movie_feature])
    W = jnp.stack([Wu, Wm]).reshape(2, 2, HALF, 256)
    b = jnp.stack([bu, bm]).reshape(2, 2, 1, HALF)
    x0s = _embed(feat, W, b)
    x0_flat = x0s.reshape(NC * NPAD, HALF)

    row = edge_index[0].astype(jnp.int32)
    col = edge_index[1].astype(jnp.int32)
    pad_idx = N_NODES + (jnp.arange(NS * E_PER_SUB - row.shape[0],
                                    dtype=jnp.int32) % (NPAD - N_NODES))
    row = jnp.concatenate([row, pad_idx])
    col = jnp.concatenate([col, pad_idx])
    er2 = jnp.stack([row, row + NPAD]).reshape(NC, NS, NSUPER, SUPER, CHUNK)
    ec3 = col.reshape(NS, NSUPER, SUPER, CHUNK)

    yk, degs = _propagate(x0_flat, er2, ec3)
    return _combine(
        x0s,
        yk.reshape(NUM_LAYERS + 1, NC, NPAD, HALF),
        degs.reshape(NC, NPAD, 1),
    )


# speedup vs baseline: 11.7657x; 1.0136x over previous
import jax
import jax.numpy as jnp
from jax import lax
from jax.experimental import pallas as pl
from jax.experimental.pallas import tpu as pltpu
from jax.experimental.pallas import tpu_sc as plsc

NUM_LAYERS = 4
N_NODES = 10000
NPAD = 10240
D = 256
HALF = 128
NS = 16
NC = 2
STRIPE = NPAD // NS
NB = STRIPE // 16
E_PER_SUB = 10240
CHUNK = 128
NCHUNK = E_PER_SUB // CHUNK
SUPER = 16
NSUPER = NCHUNK // SUPER


def _embed_body(f_ref, w_ref, b_ref, o_ref):
    o_ref[...] = (
        lax.dot_general(
            f_ref[0], w_ref[0, 0], (((1,), (1,)), ((), ())),
            preferred_element_type=jnp.float32,
        )
        + b_ref[0]
    )


def _embed(feat, W, b):
    return pl.pallas_call(
        _embed_body,
        grid=(2, 2, 5),
        in_specs=[
            pl.BlockSpec((1, 1000, 256), lambda t, c, j: (t, j, 0)),
            pl.BlockSpec((1, 1, 128, 256), lambda t, c, j: (t, c, 0, 0)),
            pl.BlockSpec((1, 1, 1, 128), lambda t, c, j: (t, c, 0, 0)),
        ],
        out_specs=pl.BlockSpec((1, 1000, 128),
                               lambda t, c, j: (c, 5 * t + j, 0)),
        out_shape=jax.ShapeDtypeStruct((NC, NPAD, HALF), jnp.float32),
    )(feat, W, b)


def _sc_body(x0_hbm, er_hbm, ec_hbm,
             yk_hbm, degs_hbm,
             acc_sh, deg_sh, rowi_v, coli_v, dinv_v, ones_v,
             zrow_v, zdeg_v, gath0_v, gath1_v, bufa0_v, bufa1_v,
             bufy0_v, bufy1_v, gsem0, gsem1, ssem0, ssem1, zsem, psem,
             asem0, asem1, wsem0, wsem1):
    c = lax.axis_index("c")
    s = lax.axis_index("s")
    my0 = s * STRIPE
    base = c * NPAD + my0
    zero16 = jnp.zeros((16,), jnp.float32)
    one16 = jnp.ones((16,), jnp.float32)
    for i in range(STRIPE // 16):
        zdeg_v[pl.ds(i * 16, 16)] = zero16
    for i in range(16):
        for j in range(HALF // 16):
            zrow_v[i, pl.ds(j * 16, 16)] = zero16
    for i in range(CHUNK // 16):
        ones_v[pl.ds(i * 16, 16)] = one16

    abufs = (bufa0_v, bufa1_v)
    asems = (asem0, asem1)
    ybufs = (bufy0_v, bufy1_v)
    wsems = (wsem0, wsem1)

    def scaled_pass(read_block, write_ref_at, square, rezero):
        def issue(k, b):
            return pltpu.async_copy(read_block(k), abufs[b], asems[b])

        issue(0, 0)
        issue(1, 1)

        def blk(k, b):
            pltpu.make_async_copy(read_block(k), abufs[b], asems[b]).wait()
            if rezero:
                pltpu.async_copy(
                    zrow_v, acc_sh.at[pl.ds(my0 + k * 16, 16)], zsem)

            @pl.when(k >= 2)
            def _():
                pltpu.make_async_copy(
                    ybufs[b], write_ref_at(k - 2), wsems[b]).wait()
            dv = dinv_v[pl.ds(k * 16, 16)]
            buf = abufs[b]
            for r in range(16):
                d = dv[r]
                if square:
                    d = d * d
                for jj in range(HALF // 16):
                    ybufs[b][r, pl.ds(jj * 16, 16)] = (
                        buf[r, pl.ds(jj * 16, 16)] * d)
            pltpu.async_copy(ybufs[b], write_ref_at(k), wsems[b])

        def body(kk, carry):
            for b in range(2):
                k = 2 * kk + b
                blk(k, b)

                @pl.when(kk < NB // 2 - 1)
                def _():
                    issue(k + 2, b)
            return carry
        lax.fori_loop(0, NB // 2, body, 0)
        for b in range(2):
            pltpu.make_async_copy(
                ybufs[b], write_ref_at(NB - 2 + b), wsems[b]).wait()
        if rezero:
            def drain(k, carry):
                pltpu.make_async_copy(
                    zrow_v, acc_sh.at[pl.ds(my0, 16)], zsem).wait()
                return carry
            lax.fori_loop(0, NB, drain, 0)

    for k in range(NB):
        pltpu.async_copy(zrow_v, acc_sh.at[pl.ds(my0 + k * 16, 16)], psem)
    pltpu.sync_copy(zdeg_v, deg_sh.at[pl.ds(my0, STRIPE)])
    plsc.subcore_barrier()

    def deg_sup(g, carry):
        pltpu.sync_copy(ec_hbm.at[s, g], coli_v)
        descs = [
            pltpu.async_copy(ones_v, deg_sh.at[coli_v.at[j]], zsem, add=True)
            for j in range(SUPER)
        ]
        for d in descs:
            d.wait()
        return carry
    lax.fori_loop(0, NSUPER, deg_sup, 0)
    plsc.subcore_barrier()

    pltpu.sync_copy(deg_sh.at[pl.ds(my0, STRIPE)], dinv_v)
    pltpu.sync_copy(dinv_v, degs_hbm.at[c, pl.ds(my0, STRIPE)])

    def newton(i, carry):
        d = dinv_v[pl.ds(i * 16, 16)]
        bits = lax.bitcast_convert_type(d, jnp.int32)
        y = lax.bitcast_convert_type(
            jnp.int32(0x5F3759DF) - (bits >> 1), jnp.float32)
        for _ in range(3):
            y = y * (1.5 - 0.5 * d * y * y)
        dinv_v[pl.ds(i * 16, 16)] = jnp.where(d > 0, y, 0.0)
        return carry
    lax.fori_loop(0, STRIPE // 16, newton, 0)

    scaled_pass(
        lambda k: x0_hbm.at[pl.ds(base + k * 16, 16)],
        lambda k: yk_hbm.at[0, pl.ds(base + k * 16, 16)],
        square=False, rezero=False,
    )
    def pdrain(k, carry):
        pltpu.make_async_copy(
            zrow_v, acc_sh.at[pl.ds(my0, 16)], psem).wait()
        return carry
    lax.fori_loop(0, NB, pdrain, 0)
    plsc.subcore_barrier()

    gbufs = (gath0_v, gath1_v)
    gsems = (gsem0, gsem1)
    ssems = (ssem0, ssem1)

    def layer(ell, carry):
        def gs_sup(g, carry2):
            pltpu.sync_copy(er_hbm.at[c, s, g], rowi_v)
            pltpu.sync_copy(ec_hbm.at[s, g], coli_v)
            src = yk_hbm.at[ell]
            gd = [None, None]
            sd = [None, None]
            gd[0] = pltpu.async_copy(src.at[rowi_v.at[0]], gbufs[0],
                                     gsems[0])
            for j in range(SUPER):
                cur = j % 2
                oth = 1 - cur
                if j + 1 < SUPER:
                    if sd[oth] is not None:
                        sd[oth].wait()
                    gd[oth] = pltpu.async_copy(
                        src.at[rowi_v.at[j + 1]], gbufs[oth], gsems[oth])
                gd[cur].wait()
                sd[cur] = pltpu.async_copy(
                    gbufs[cur], acc_sh.at[coli_v.at[j]], ssems[cur], add=True)
            sd[0].wait()
            sd[1].wait()
            return carry2
        lax.fori_loop(0, NSUPER, gs_sup, 0)
        plsc.subcore_barrier()

        scaled_pass(
            lambda k: acc_sh.at[pl.ds(my0 + k * 16, 16)],
            lambda k: yk_hbm.at[ell + 1, pl.ds(base + k * 16, 16)],
            square=True, rezero=True,
        )
        plsc.subcore_barrier()
        return carry

    lax.fori_loop(0, NUM_LAYERS, layer, 0)


def _propagate(x0_flat, er2, ec3):
    mesh = plsc.VectorSubcoreMesh(
        core_axis_name="c", subcore_axis_name="s",
        num_cores=NC, num_subcores=NS,
    )
    return pl.kernel(
        _sc_body,
        out_type=[
            jax.ShapeDtypeStruct((NUM_LAYERS + 1, NC * NPAD, HALF),
                                 jnp.float32),
            jax.ShapeDtypeStruct((NC, NPAD), jnp.float32),
        ],
        mesh=mesh,
        scratch_types=[
            pltpu.VMEM_SHARED((NPAD, HALF), jnp.float32),
            pltpu.VMEM_SHARED((NPAD,), jnp.float32),
            pltpu.VMEM((SUPER, CHUNK), jnp.int32),
            pltpu.VMEM((SUPER, CHUNK), jnp.int32),
            pltpu.VMEM((STRIPE,), jnp.float32),
            pltpu.VMEM((CHUNK,), jnp.float32),
            pltpu.VMEM((16, HALF), jnp.float32),
            pltpu.VMEM((STRIPE,), jnp.float32),
            pltpu.VMEM((CHUNK, HALF), jnp.float32),
            pltpu.VMEM((CHUNK, HALF), jnp.float32),
            pltpu.VMEM((16, HALF), jnp.float32),
            pltpu.VMEM((16, HALF), jnp.float32),
            pltpu.VMEM((16, HALF), jnp.float32),
            pltpu.VMEM((16, HALF), jnp.float32),
            pltpu.SemaphoreType.DMA,
            pltpu.SemaphoreType.DMA,
            pltpu.SemaphoreType.DMA,
            pltpu.SemaphoreType.DMA,
            pltpu.SemaphoreType.DMA,
            pltpu.SemaphoreType.DMA,
            pltpu.SemaphoreType.DMA,
            pltpu.SemaphoreType.DMA,
            pltpu.SemaphoreType.DMA,
            pltpu.SemaphoreType.DMA,
        ],
    )(x0_flat, er2, ec3)


def _combine_body(x0r, ykr, degr, o_ref):
    sq = jnp.sqrt(degr[0])
    o_ref[...] = x0r[0] + sq * (
        0.5 * ykr[1, 0] + (1.0 / 3.0) * ykr[2, 0]
        + 0.25 * ykr[3, 0] + 0.2 * ykr[4, 0]
    )


def _combine(x0s, yk, degs):
    return pl.pallas_call(
        _combine_body,
        grid=(2, 10),
        in_specs=[
            pl.BlockSpec((1, 1000, HALF), lambda c, i: (c, i, 0)),
            pl.BlockSpec((NUM_LAYERS + 1, 1, 1000, HALF),
                         lambda c, i: (0, c, i, 0)),
            pl.BlockSpec((1, 1000, 1), lambda c, i: (c, i, 0)),
        ],
        out_specs=pl.BlockSpec((1000, HALF), lambda c, i: (i, c)),
        out_shape=jax.ShapeDtypeStruct((N_NODES, D), jnp.float32),
    )(x0s, yk, degs)


def kernel(user_feature, movie_feature, edge_index, Wu, bu, Wm, bm):
    feat = jnp.stack([user_feature, movie_feature])
    W = jnp.stack([Wu, Wm]).reshape(2, 2, HALF, 256)
    b = jnp.stack([bu, bm]).reshape(2, 2, 1, HALF)
    x0s = _embed(feat, W, b)
    x0_flat = x0s.reshape(NC * NPAD, HALF)

    row = edge_index[0].astype(jnp.int32)
    col = edge_index[1].astype(jnp.int32)
    pad_idx = N_NODES + (jnp.arange(NS * E_PER_SUB - row.shape[0],
                                    dtype=jnp.int32) % (NPAD - N_NODES))
    row = jnp.concatenate([row, pad_idx])
    col = jnp.concatenate([col, pad_idx])
    er2 = jnp.stack([row, row + NPAD]).reshape(NC, NS, NSUPER, SUPER, CHUNK)
    ec3 = col.reshape(NS, NSUPER, SUPER, CHUNK)

    yk, degs = _propagate(x0_flat, er2, ec3)
    return _combine(
        x0s,
        yk.reshape(NUM_LAYERS + 1, NC, NPAD, HALF),
        degs.reshape(NC, NPAD, 1),
    )
